# baseline (speedup 1.0000x reference)
import functools

import jax
import jax.numpy as jnp
from jax import lax
from jax.experimental import pallas as pl
from jax.experimental.pallas import tpu as pltpu
from jax.experimental.pallas import tpu_sc as plsc

_T = 100000
_D = 64
_B = 16384

_NC = 2
_NS = 16
_NW = _NC * _NS
_B_PER_W = _B // _NW
_CHUNK = 128
_NCHUNK = _B_PER_W // _CHUNK
_L = 16

_mesh = plsc.VectorSubcoreMesh(core_axis_name="c", subcore_axis_name="s")


@functools.partial(
    pl.kernel,
    mesh=_mesh,
    out_type=jax.ShapeDtypeStruct((_B, 2 * _D), jnp.float32),
    scratch_types=[
        pltpu.VMEM((_B_PER_W,), jnp.int32),
        pltpu.VMEM((_B_PER_W,), jnp.int32),
        pltpu.VMEM((_B_PER_W, 2 * _D), jnp.float32),
        pltpu.SemaphoreType.DMA,
    ],
)
def _pe_gather_wide(pe2_hbm, x_hbm, wide_hbm, idx_v, hi_v, rows_v, sem):
    wid = lax.axis_index("s") * _NC + lax.axis_index("c")
    base = wid * _B_PER_W
    pltpu.sync_copy(x_hbm.at[pl.ds(base, _B_PER_W)], idx_v)

    for b in range(_B_PER_W // _L):
        idx16 = idx_v[pl.ds(b * _L, _L)]
        hi_v[pl.ds(b * _L, _L)] = lax.shift_right_logical(idx16, 1)

    copies = [
        pltpu.async_copy(
            pe2_hbm.at[hi_v.at[pl.ds(c * _CHUNK, _CHUNK)]],
            rows_v.at[pl.ds(c * _CHUNK, _CHUNK)],
            sem,
        )
        for c in range(_NCHUNK)
    ]
    for c, cp in enumerate(copies):
        cp.wait()
        pltpu.sync_copy(
            rows_v.at[pl.ds(c * _CHUNK, _CHUNK)],
            wide_hbm.at[pl.ds(base + c * _CHUNK, _CHUNK)],
        )


_ROWS_TC = 1024


def _select_body(x_ref, wide_ref, out_ref):
    half = x_ref[0] & 1
    out_ref[...] = jnp.where(
        half == 0, wide_ref[:, : _D], wide_ref[:, _D:]
    )


_select = pl.pallas_call(
    _select_body,
    grid=(_B // _ROWS_TC,),
    in_specs=[
        pl.BlockSpec((1, _ROWS_TC, 1), lambda i: (i, 0, 0)),
        pl.BlockSpec((_ROWS_TC, 2 * _D), lambda i: (i, 0)),
    ],
    out_specs=pl.BlockSpec((_ROWS_TC, _D), lambda i: (i, 0)),
    out_shape=jax.ShapeDtypeStruct((_B, _D), jnp.float32),
)


def kernel(x, pe):
    xi = x.astype(jnp.int32)
    pe2 = pe.reshape(_T // 2, 2 * _D)
    wide = _pe_gather_wide(pe2, xi)
    return _select(xi.reshape(_B // _ROWS_TC, _ROWS_TC, 1), wide)


# speedup vs baseline: 2.1224x; 2.1224x over previous
import functools

import jax
import jax.numpy as jnp
from jax import lax
from jax.experimental import pallas as pl
from jax.experimental.pallas import tpu as pltpu
from jax.experimental.pallas import tpu_sc as plsc

_T = 100000
_D = 64
_B = 16384

_NC = 2
_NS = 16
_NW = _NC * _NS
_DIMS_PER_W = _D // _NW
_XCHUNK = 4096
_NXCHUNK = _B // _XCHUNK
_L = 16

_mesh = plsc.VectorSubcoreMesh(core_axis_name="c", subcore_axis_name="s")


@functools.partial(
    pl.kernel,
    mesh=_mesh,
    compiler_params=pltpu.CompilerParams(needs_layout_passes=False),
    out_type=jax.ShapeDtypeStruct((_D, _B), jnp.float32),
    scratch_types=[
        pltpu.VMEM((_T,), jnp.float32),
        pltpu.VMEM((_XCHUNK,), jnp.int32),
        pltpu.VMEM((_XCHUNK,), jnp.float32),
        pltpu.SemaphoreType.DMA,
    ],
)
def _pe_gather_t(pet_hbm, x_hbm, outt_hbm, row_v, xc_v, oc_v, sem):
    wid = lax.axis_index("s") * _NC + lax.axis_index("c")

    for k in range(_DIMS_PER_W):
        d = wid * _DIMS_PER_W + k
        pltpu.sync_copy(pet_hbm.at[d], row_v)
        for q in range(_NXCHUNK):
            pltpu.sync_copy(x_hbm.at[pl.ds(q * _XCHUNK, _XCHUNK)], xc_v)

            def gather_block(b, _):
                idx16 = xc_v[pl.ds(b * _L, _L)]
                oc_v[pl.ds(b * _L, _L)] = plsc.load_gather(row_v, [idx16])
                return _

            lax.fori_loop(0, _XCHUNK // _L, gather_block, None)
            pltpu.sync_copy(oc_v, outt_hbm.at[d, pl.ds(q * _XCHUNK, _XCHUNK)])


def kernel(x, pe):
    outt = _pe_gather_t(pe.T, x.astype(jnp.int32))
    return outt.T


# speedup vs baseline: 2.6868x; 1.2659x over previous
import functools

import jax
import jax.numpy as jnp
from jax import lax
from jax.experimental import pallas as pl
from jax.experimental.pallas import tpu as pltpu
from jax.experimental.pallas import tpu_sc as plsc

_T = 100000
_D = 64
_B = 16384

_NC = 2
_NS = 16
_NW = _NC * _NS
_DIMS_PER_W = _D // _NW
_XCHUNK = 4096
_NXCHUNK = _B // _XCHUNK
_L = 16
_UNROLL = 8

_mesh = plsc.VectorSubcoreMesh(core_axis_name="c", subcore_axis_name="s")


@functools.partial(
    pl.kernel,
    mesh=_mesh,
    compiler_params=pltpu.CompilerParams(needs_layout_passes=False),
    out_type=jax.ShapeDtypeStruct((_D, _B), jnp.float32),
    scratch_types=[
        pltpu.VMEM((_T,), jnp.float32),
        pltpu.VMEM((_B,), jnp.int32),
        pltpu.VMEM((_XCHUNK,), jnp.float32),
        pltpu.VMEM((_XCHUNK,), jnp.float32),
        pltpu.SemaphoreType.DMA,
        pltpu.SemaphoreType.DMA,
        pltpu.SemaphoreType.DMA,
        pltpu.SemaphoreType.DMA,
    ],
)
def _pe_gather_t(pet_hbm, x_hbm, outt_hbm, row_v, x_v, oc0_v, oc1_v,
                 rsem, xsem, wsem0, wsem1):
    wid = lax.axis_index("s") * _NC + lax.axis_index("c")
    ocs = (oc0_v, oc1_v)
    wsems = (wsem0, wsem1)

    xcp = pltpu.async_copy(x_hbm, x_v, xsem)
    pending = [None, None]
    for k in range(_DIMS_PER_W):
        d = wid * _DIMS_PER_W + k
        rcp = pltpu.async_copy(pet_hbm.at[d], row_v, rsem)
        rcp.wait()
        if k == 0:
            xcp.wait()
        for q in range(_NXCHUNK):
            buf = q % 2
            oc_v = ocs[buf]
            if pending[buf] is not None:
                pending[buf].wait()

            def gather_block(b, _):
                base = q * _XCHUNK + b * (_L * _UNROLL)
                for j in range(_UNROLL):
                    idx16 = x_v[pl.ds(base + j * _L, _L)]
                    oc_v[pl.ds(b * (_L * _UNROLL) + j * _L, _L)] = (
                        plsc.load_gather(row_v, [idx16])
                    )
                return _

            lax.fori_loop(0, _XCHUNK // (_L * _UNROLL), gather_block, None)
            pending[buf] = pltpu.async_copy(
                oc_v, outt_hbm.at[d, pl.ds(q * _XCHUNK, _XCHUNK)], wsems[buf]
            )
    for cp in pending:
        if cp is not None:
            cp.wait()


def kernel(x, pe):
    outt = _pe_gather_t(pe.T, x.astype(jnp.int32))
    return outt.T
